# baseline (speedup 1.0000x reference)
import functools

import jax
import jax.numpy as jnp
from jax import lax
from jax.experimental import pallas as pl
from jax.experimental.pallas import tpu as pltpu
from jax.experimental.pallas import tpu_sc as plsc

B = 16384
D = 32
NUMF = 10
HID = 128
NC, NS = 2, 16
NW = NC * NS
BPW = B // NW
CHUNK = 128
NCHUNK = BPW // CHUNK


def _sc_gather(uids, mids, dids, user_table, merchant_table, device_table):
    mesh = plsc.VectorSubcoreMesh(
        core_axis_name="c", subcore_axis_name="s", num_cores=NC, num_subcores=NS
    )
    row_shape = jax.ShapeDtypeStruct((B // CHUNK, CHUNK, D), jnp.float32)

    @functools.partial(
        pl.kernel,
        out_type=(row_shape, row_shape, row_shape),
        mesh=mesh,
        scratch_types=[
            pltpu.VMEM((NCHUNK, CHUNK), jnp.int32),
            pltpu.VMEM((NCHUNK, CHUNK), jnp.int32),
            pltpu.VMEM((NCHUNK, CHUNK), jnp.int32),
            pltpu.VMEM((NCHUNK, CHUNK, D), jnp.float32),
            pltpu.VMEM((NCHUNK, CHUNK, D), jnp.float32),
            pltpu.VMEM((NCHUNK, CHUNK, D), jnp.float32),
            pltpu.SemaphoreType.DMA,
        ],
        compiler_params=pltpu.CompilerParams(use_tc_tiling_on_sc=False),
    )
    def k(uids_hbm, mids_hbm, dids_hbm, ut_hbm, mt_hbm, dt_hbm,
          out_u, out_m, out_d, idx_u, idx_m, idx_d, rows_u, rows_m, rows_d,
          sem):
        wid = lax.axis_index("s") * NC + lax.axis_index("c")
        row0 = wid * NCHUNK
        pltpu.sync_copy(uids_hbm.at[pl.ds(row0, NCHUNK)], idx_u)
        pltpu.sync_copy(mids_hbm.at[pl.ds(row0, NCHUNK)], idx_m)
        pltpu.sync_copy(dids_hbm.at[pl.ds(row0, NCHUNK)], idx_d)
        copies = []
        for j in range(NCHUNK):
            copies.append(pltpu.async_copy(ut_hbm.at[idx_u.at[j]], rows_u.at[j], sem))
            copies.append(pltpu.async_copy(mt_hbm.at[idx_m.at[j]], rows_m.at[j], sem))
            copies.append(pltpu.async_copy(dt_hbm.at[idx_d.at[j]], rows_d.at[j], sem))
        for c in copies:
            c.wait()
        pltpu.sync_copy(rows_u, out_u.at[pl.ds(row0, NCHUNK)])
        pltpu.sync_copy(rows_m, out_m.at[pl.ds(row0, NCHUNK)])
        pltpu.sync_copy(rows_d, out_d.at[pl.ds(row0, NCHUNK)])

    return k(uids, mids, dids, user_table, merchant_table, device_table)


BLK = 2048


def _mlp_body(u_ref, m_ref, d_ref, nf_ref, w1a_ref, w1b_ref, b1_ref,
              w2_ref, b2_ref, o_ref):
    e = (u_ref[...] + m_ref[...] + d_ref[...]) * (1.0 / 3.0)
    h = jnp.dot(e, w1a_ref[...], preferred_element_type=jnp.float32)
    h = h + jnp.dot(nf_ref[...], w1b_ref[...], preferred_element_type=jnp.float32)
    h = jnp.maximum(h + b1_ref[...], 0.0)
    out = jnp.dot(h, w2_ref[...], preferred_element_type=jnp.float32) + b2_ref[...]
    norm = jnp.sqrt(jnp.sum(out * out, axis=1, keepdims=True))
    o_ref[...] = out / jnp.maximum(norm, 1e-12)


def _mlp(u, m, d, nf, w1a, w1b, b1, w2, b2):
    full = lambda shape: pl.BlockSpec(shape, lambda i: (0, 0))
    return pl.pallas_call(
        _mlp_body,
        grid=(B // BLK,),
        in_specs=[
            pl.BlockSpec((BLK, D), lambda i: (i, 0)),
            pl.BlockSpec((BLK, D), lambda i: (i, 0)),
            pl.BlockSpec((BLK, D), lambda i: (i, 0)),
            pl.BlockSpec((BLK, NUMF), lambda i: (i, 0)),
            full((D, HID)),
            full((NUMF, HID)),
            full((1, HID)),
            full((HID, HID)),
            full((1, HID)),
        ],
        out_specs=pl.BlockSpec((BLK, HID), lambda i: (i, 0)),
        out_shape=jax.ShapeDtypeStruct((B, HID), jnp.float32),
    )(u, m, d, nf, w1a, w1b, b1, w2, b2)


def kernel(user_ids, merchant_ids, device_ids, numerical_features,
           user_table, merchant_table, device_table, W1, b1, W2, b2):
    uids = user_ids.reshape(B // CHUNK, CHUNK)
    mids = merchant_ids.reshape(B // CHUNK, CHUNK)
    dids = device_ids.reshape(B // CHUNK, CHUNK)
    u_rows, m_rows, d_rows = _sc_gather(
        uids, mids, dids, user_table, merchant_table, device_table)
    out = _mlp(
        u_rows.reshape(B, D), m_rows.reshape(B, D), d_rows.reshape(B, D),
        numerical_features,
        W1[:D], W1[D:], b1.reshape(1, HID), W2, b2.reshape(1, HID))
    return out


# speedup vs baseline: 1.0247x; 1.0247x over previous
import functools

import jax
import jax.numpy as jnp
from jax import lax
from jax.experimental import pallas as pl
from jax.experimental.pallas import tpu as pltpu
from jax.experimental.pallas import tpu_sc as plsc

B = 16384
D = 32
WIDE = 128
NUMF = 10
HID = 128
NC, NS = 2, 16
NW = NC * NS
BPW = B // NW
CHUNK = 128
NCHUNK = BPW // CHUNK


def _sc_gather(uids, mids, dids, user_table, merchant_table, device_table):
    mesh = plsc.VectorSubcoreMesh(
        core_axis_name="c", subcore_axis_name="s", num_cores=NC, num_subcores=NS
    )
    out_shape = jax.ShapeDtypeStruct((B, WIDE), jnp.float32)

    @functools.partial(
        pl.kernel,
        out_type=(out_shape, out_shape, out_shape),
        mesh=mesh,
        scratch_types=[
            pltpu.VMEM((NCHUNK, CHUNK), jnp.int32),
            pltpu.VMEM((NCHUNK, CHUNK), jnp.int32),
            pltpu.VMEM((NCHUNK, CHUNK), jnp.int32),
            pltpu.VMEM((BPW, D), jnp.float32),
            pltpu.VMEM((BPW, D), jnp.float32),
            pltpu.VMEM((BPW, D), jnp.float32),
            pltpu.SemaphoreType.DMA,
        ],
        compiler_params=pltpu.CompilerParams(use_tc_tiling_on_sc=False),
    )
    def k(uids_hbm, mids_hbm, dids_hbm, ut_hbm, mt_hbm, dt_hbm,
          out_u, out_m, out_d, idx_u, idx_m, idx_d, rows_u, rows_m, rows_d,
          sem):
        wid = lax.axis_index("s") * NC + lax.axis_index("c")
        base = wid * BPW
        for j in range(NCHUNK):
            pltpu.sync_copy(uids_hbm.at[pl.ds(base + j * CHUNK, CHUNK)], idx_u.at[j])
            pltpu.sync_copy(mids_hbm.at[pl.ds(base + j * CHUNK, CHUNK)], idx_m.at[j])
            pltpu.sync_copy(dids_hbm.at[pl.ds(base + j * CHUNK, CHUNK)], idx_d.at[j])
        copies = []
        for j in range(NCHUNK):
            copies.append(pltpu.async_copy(
                ut_hbm.at[idx_u.at[j]], rows_u.at[pl.ds(j * CHUNK, CHUNK)], sem))
            copies.append(pltpu.async_copy(
                mt_hbm.at[idx_m.at[j]], rows_m.at[pl.ds(j * CHUNK, CHUNK)], sem))
            copies.append(pltpu.async_copy(
                dt_hbm.at[idx_d.at[j]], rows_d.at[pl.ds(j * CHUNK, CHUNK)], sem))
        for c in copies:
            c.wait()
        pltpu.sync_copy(rows_u, out_u.at[pl.ds(base, BPW), pl.ds(0, D)])
        pltpu.sync_copy(rows_m, out_m.at[pl.ds(base, BPW), pl.ds(0, D)])
        pltpu.sync_copy(rows_d, out_d.at[pl.ds(base, BPW), pl.ds(0, D)])

    return k(uids, mids, dids, user_table, merchant_table, device_table)


BLK = 2048


def _mlp_body(u_ref, m_ref, d_ref, nf_ref, w1a_ref, w1b_ref, b1_ref,
              w2_ref, b2_ref, o_ref):
    e = (u_ref[:, :D] + m_ref[:, :D] + d_ref[:, :D]) * (1.0 / 3.0)
    h = jnp.dot(e, w1a_ref[...], preferred_element_type=jnp.float32)
    h = h + jnp.dot(nf_ref[...], w1b_ref[...], preferred_element_type=jnp.float32)
    h = jnp.maximum(h + b1_ref[...], 0.0)
    out = jnp.dot(h, w2_ref[...], preferred_element_type=jnp.float32) + b2_ref[...]
    norm = jnp.sqrt(jnp.sum(out * out, axis=1, keepdims=True))
    o_ref[...] = out / jnp.maximum(norm, 1e-12)


def _mlp(u, m, d, nf, w1a, w1b, b1, w2, b2):
    full = lambda shape: pl.BlockSpec(shape, lambda i: (0, 0))
    return pl.pallas_call(
        _mlp_body,
        grid=(B // BLK,),
        in_specs=[
            pl.BlockSpec((BLK, WIDE), lambda i: (i, 0)),
            pl.BlockSpec((BLK, WIDE), lambda i: (i, 0)),
            pl.BlockSpec((BLK, WIDE), lambda i: (i, 0)),
            pl.BlockSpec((BLK, NUMF), lambda i: (i, 0)),
            full((D, HID)),
            full((NUMF, HID)),
            full((1, HID)),
            full((HID, HID)),
            full((1, HID)),
        ],
        out_specs=pl.BlockSpec((BLK, HID), lambda i: (i, 0)),
        out_shape=jax.ShapeDtypeStruct((B, HID), jnp.float32),
    )(u, m, d, nf, w1a, w1b, b1, w2, b2)


def kernel(user_ids, merchant_ids, device_ids, numerical_features,
           user_table, merchant_table, device_table, W1, b1, W2, b2):
    u_rows, m_rows, d_rows = _sc_gather(
        user_ids, merchant_ids, device_ids,
        user_table, merchant_table, device_table)
    out = _mlp(
        u_rows, m_rows, d_rows, numerical_features,
        W1[:D], W1[D:], b1.reshape(1, HID), W2, b2.reshape(1, HID))
    return out
